# baseline (speedup 1.0000x reference)
import functools

import jax
import jax.numpy as jnp
from jax import lax
from jax.experimental import pallas as pl
from jax.experimental.pallas import tpu as pltpu
from jax.experimental.pallas import tpu_sc as plsc

GAMMA_ = 12.0
B_ = 16384
D_ = 64
NC_ = 2
NS_ = 16
NW_ = NC_ * NS_
PER_W_ = B_ // NW_
CHUNK_ = 128
NCH_ = PER_W_ // CHUNK_


_GATHER_DNUMS = lax.GatherDimensionNumbers(
    offset_dims=(), collapsed_slice_dims=(0,), start_index_map=(0,))


def _lane_shuffle(x, idx):
    return lax.gather(
        x, idx[:, None], _GATHER_DNUMS, (1,),
        indices_are_sorted=False, unique_indices=False,
        mode=lax.GatherScatterMode.PROMISE_IN_BOUNDS)


def _build():
    mesh = plsc.VectorSubcoreMesh(core_axis_name="c", subcore_axis_name="s")

    @functools.partial(
        pl.kernel,
        mesh=mesh,
        compiler_params=pltpu.CompilerParams(use_tc_tiling_on_sc=False),
        out_type=jax.ShapeDtypeStruct((B_,), jnp.float32),
        scratch_types=[
            pltpu.VMEM((NCH_, CHUNK_), jnp.int32),
            pltpu.VMEM((NCH_, CHUNK_), jnp.int32),
            pltpu.VMEM((NCH_, CHUNK_), jnp.int32),
            pltpu.VMEM((CHUNK_, D_), jnp.float32),
            pltpu.VMEM((CHUNK_, D_), jnp.float32),
            pltpu.VMEM((CHUNK_, D_), jnp.float32),
            pltpu.VMEM((PER_W_,), jnp.float32),
            pltpu.SemaphoreType.DMA,
        ],
    )
    def k(hidx_hbm, ridx_hbm, tidx_hbm, ent_hbm, rel_hbm, out_hbm,
          hidx_v, ridx_v, tidx_v, hv, rv, tv, out_v, sem):
        wid = lax.axis_index("s") * NC_ + lax.axis_index("c")

        pltpu.sync_copy(hidx_hbm.at[wid], hidx_v)
        pltpu.sync_copy(ridx_hbm.at[wid], ridx_v)
        pltpu.sync_copy(tidx_hbm.at[wid], tidx_v)

        for c in range(NCH_):
            cp_h = pltpu.async_copy(ent_hbm.at[hidx_v.at[c]], hv, sem)
            cp_r = pltpu.async_copy(rel_hbm.at[ridx_v.at[c]], rv, sem)
            cp_t = pltpu.async_copy(ent_hbm.at[tidx_v.at[c]], tv, sem)
            cp_h.wait()
            cp_r.wait()
            cp_t.wait()

            def body(g, _, c=c):
                lane = lax.iota(jnp.int32, 16)
                packed = jnp.zeros((16,), jnp.float32)
                for j in range(16):
                    s = g * 16 + j
                    acc = jnp.zeros((16,), jnp.float32)
                    for db in range(D_ // 16):
                        sl = pl.ds(db * 16, 16)
                        acc = acc + jnp.abs(hv[s, sl] + rv[s, sl] - tv[s, sl])
                    for k in (1, 2, 4, 8):
                        acc = acc + _lane_shuffle(acc, lane ^ k)
                    packed = jnp.where(lane == j, GAMMA_ - acc, packed)
                out_v[pl.ds(c * CHUNK_ + g * 16, 16)] = packed
                return 0

            lax.fori_loop(0, CHUNK_ // 16, body, 0)

        pltpu.sync_copy(out_v, out_hbm.at[pl.ds(wid * PER_W_, PER_W_)])

    return k


_score_kernel = _build()


def kernel(sample, entity_embedding, relation_embedding):
    sample = sample.astype(jnp.int32)
    hidx = sample[:, 0].reshape(NW_, NCH_, CHUNK_)
    ridx = sample[:, 1].reshape(NW_, NCH_, CHUNK_)
    tidx = sample[:, 2].reshape(NW_, NCH_, CHUNK_)
    out = _score_kernel(hidx, ridx, tidx, entity_embedding, relation_embedding)
    return out.reshape(B_, 1)
